# baseline (speedup 1.0000x reference)
import functools

import jax
import jax.numpy as jnp
from jax import lax
from jax.experimental import pallas as pl
from jax.experimental.pallas import tpu as pltpu
from jax.experimental.pallas import tpu_sc as plsc

_OHEM_RATIO = 0.7
_IGNORE_INDEX = -100
_EPS = 1e-07

_NBINS = 32768
_NW = 32


def _loss_body(pred_ref, tgt_ref, loss_ref, *, hb):
    C = pred_ref.shape[1]
    W = pred_ref.shape[3]
    p = pred_ref[0]
    t = tgt_ref[0]
    tcl = jnp.clip(t, 0, C - 1)
    cls = lax.broadcasted_iota(jnp.int32, (C, hb, W), 0)
    e = jnp.exp(p)
    g = jnp.where(cls == tcl[None], p, 0.0)

    def _tree(planes):
        while len(planes) > 1:
            nxt = [planes[a] + planes[a + 1]
                   for a in range(0, len(planes) - 1, 2)]
            if len(planes) % 2:
                nxt.append(planes[-1])
            planes = nxt
        return planes[0]

    s = _tree([e[i] for i in range(C)])
    pt = _tree([g[i] for i in range(C)])
    nll = jnp.maximum(jnp.log(s) - pt, 0.0)
    nll = jnp.where(t == _IGNORE_INDEX, 0.0, nll)
    loss_ref[...] = lax.bitcast_convert_type(nll, jnp.int32)


def _sc_hist(n):
    chunk = n // _NW
    mesh = plsc.VectorSubcoreMesh(core_axis_name="c", subcore_axis_name="s")

    @functools.partial(
        pl.kernel, mesh=mesh,
        out_type=jax.ShapeDtypeStruct((_NW, _NBINS), jnp.int32),
        compiler_params=pltpu.CompilerParams(needs_layout_passes=False),
        scratch_types=[
            pltpu.VMEM((chunk,), jnp.int32),
            pltpu.VMEM((_NBINS,), jnp.int32),
        ],
    )
    def run(loss_hbm, out_hbm, chunk_v, hist_v):
        wid = lax.axis_index("s") * 2 + lax.axis_index("c")
        base = wid * chunk
        zero16 = jnp.zeros((16,), jnp.int32)

        def z(i, c):
            hist_v[pl.ds(i * 16, 16)] = zero16
            return c
        lax.fori_loop(0, _NBINS // 16, z, 0)
        pltpu.sync_copy(loss_hbm.at[pl.ds(base, chunk)], chunk_v)
        one16 = jnp.ones((16,), jnp.int32)

        def body(i, c):
            kx = chunk_v[pl.ds(i * 16, 16)]
            idx = lax.shift_right_logical(kx, 16)
            plsc.addupdate_scatter(hist_v, [idx], one16)
            return c
        lax.fori_loop(0, chunk // 16, body, 0)
        pltpu.sync_copy(hist_v, out_hbm.at[wid])

    return run


def _select_body(loss_ref, hist_ref, out_ref, mlo_ref, *, k):
    n_rows, W = loss_ref.shape
    hr, hl = hist_ref.shape[1], hist_ref.shape[2]

    def merge(i, acc):
        return acc + hist_ref[i]
    hsum = lax.fori_loop(0, _NW, merge,
                         jnp.zeros((hr, hl), jnp.int32))
    bin_iota = (lax.broadcasted_iota(jnp.int32, (hr, hl), 0) * hl
                + lax.broadcasted_iota(jnp.int32, (hr, hl), 1))

    def count_hi(thr):
        return jnp.sum(jnp.where(bin_iota >= thr, hsum, 0))

    def bs_hi(i, state):
        lo, hi, c_lo, c_hi = state
        mid = lo + (hi - lo) // 2
        c = count_hi(mid)
        ge = c >= k
        return (jnp.where(ge, mid, lo), jnp.where(ge, hi, mid),
                jnp.where(ge, c, c_lo), jnp.where(ge, c_hi, c))

    p_star, _, c_lo, c_gt = lax.fori_loop(
        0, 15, bs_hi,
        (jnp.int32(0), jnp.int32(_NBINS), jnp.int32(n_rows * W),
         jnp.int32(0)))
    k2 = k - c_gt

    def prep(i, carry):
        for u in range(4):
            base = (i * 4 + u) * 32
            kx = loss_ref[pl.ds(base, 32), :]
            inb = (kx >> 16) == p_star
            m = jnp.where(inb, kx & 0xFFFF, 0) - 32768
            mlo_ref[pl.ds(base, 32), :] = m.astype(jnp.int16)
        return carry
    lax.fori_loop(0, n_rows // 128, prep, jnp.int32(0))

    def count_lo(q):
        q16 = (q - 32768).astype(jnp.int16)

        def body(i, acc):
            for u in range(4):
                x = mlo_ref[pl.ds((i * 4 + u) * 64, 64), :]
                acc = acc + (x >= q16).astype(jnp.int16)
            return acc
        acc = lax.fori_loop(0, n_rows // 256, body,
                            jnp.zeros((64, W), jnp.int16))
        return jnp.sum(acc.astype(jnp.int32))

    def bs_lo(i, state):
        lo, hi = state
        mid = lo + (hi - lo) // 2
        ge = count_lo(mid) >= k2
        return (jnp.where(ge, mid, lo), jnp.where(ge, hi, mid))

    q_star, _ = lax.fori_loop(0, 16, bs_lo, (jnp.int32(0), jnp.int32(65536)))
    thresh = (p_star << 16) | q_star

    def final(i, st):
        sa, ca = st
        for u in range(4):
            base = (i * 4 + u) * 32
            kx = loss_ref[pl.ds(base, 32), :]
            x = lax.bitcast_convert_type(kx, jnp.float32)
            msk = kx >= thresh
            sa = sa + jnp.where(msk, x, 0.0)
            ca = ca + msk.astype(jnp.int32)
        return sa, ca
    sa, ca = lax.fori_loop(
        0, n_rows // 128, final,
        (jnp.zeros((32, W), jnp.float32), jnp.zeros((32, W), jnp.int32)))
    total = jnp.sum(sa)
    cnt = jnp.sum(ca).astype(jnp.float32)
    out_ref[0, 0] = total / (cnt + _EPS)


def kernel(pred, target):
    B, C, H, W = pred.shape
    hb = 128 if H % 128 == 0 else 8
    nb = B * (H // hb)
    n_rows = B * H
    n = n_rows * W
    k = int(n * _OHEM_RATIO)
    target = target.astype(jnp.int32)
    hpb = H // hb

    loss = pl.pallas_call(
        functools.partial(_loss_body, hb=hb),
        grid=(nb,),
        in_specs=[
            pl.BlockSpec((1, C, hb, W), lambda i: (i // hpb, 0, i % hpb, 0)),
            pl.BlockSpec((1, hb, W), lambda i: (i // hpb, i % hpb, 0)),
        ],
        out_specs=pl.BlockSpec((hb, W), lambda i: (i, 0)),
        out_shape=jax.ShapeDtypeStruct((n_rows, W), jnp.int32),
    )(pred, target)

    hists = _sc_hist(n)(loss.reshape(n))
    hists = hists.reshape(_NW, _NBINS // 128, 128)

    out = pl.pallas_call(
        functools.partial(_select_body, k=k),
        grid=(1,),
        in_specs=[
            pl.BlockSpec((n_rows, W), lambda i: (0, 0)),
            pl.BlockSpec((_NW, _NBINS // 128, 128), lambda i: (0, 0, 0)),
        ],
        out_specs=pl.BlockSpec(memory_space=pltpu.SMEM),
        out_shape=jax.ShapeDtypeStruct((1, 1), jnp.float32),
        scratch_shapes=[pltpu.VMEM((n_rows, W), jnp.int16)],
    )(loss, hists)
    return out[0, 0]
